# baseline (speedup 1.0000x reference)
import jax
import jax.numpy as jnp
from jax.experimental import pallas as pl

H_ = 10


def _proj_attn_kernel(x_ref, w1_ref, aw_ref, h_ref, al_ref):
    h = jnp.dot(x_ref[...], w1_ref[...], preferred_element_type=jnp.float32)
    h_ref[...] = h
    al_ref[...] = jnp.dot(h, aw_ref[...], preferred_element_type=jnp.float32)


def _gcn_mm_kernel(g_ref, b1_ref, w2_ref, out_ref):
    hin = jnp.maximum(g_ref[...] + b1_ref[...], 0.0)
    out_ref[...] = jnp.dot(hin, w2_ref[...], preferred_element_type=jnp.float32)


def _mlp_kernel(g_ref, wf1_ref, bf1_ref, wf2_ref, bf2_ref, out_ref):
    t = jnp.dot(g_ref[...], wf1_ref[...], preferred_element_type=jnp.float32)
    t = jnp.maximum(t + bf1_ref[...], 0.0)
    out_ref[...] = (
        jnp.dot(t, wf2_ref[...], preferred_element_type=jnp.float32) + bf2_ref[...]
    )


def kernel(x, edge_index, batch, W1, a_src, a_dst, b1, W2, b2, Wf1, bf1, Wf2, bf2):
    n, f = x.shape
    HC = W1.shape[1]
    C = HC // H_
    nG = 256

    eye = jnp.eye(H_, dtype=x.dtype)
    As = (a_src[:, :, None] * eye[:, None, :]).reshape(HC, H_)
    Ad = (a_dst[:, :, None] * eye[:, None, :]).reshape(HC, H_)
    AW = jnp.concatenate([As, Ad], axis=1)

    BLK = 1000
    grid = (n // BLK,)
    h, al = pl.pallas_call(
        _proj_attn_kernel,
        grid=grid,
        in_specs=[
            pl.BlockSpec((BLK, f), lambda i: (i, 0)),
            pl.BlockSpec((f, HC), lambda i: (0, 0)),
            pl.BlockSpec((HC, 2 * H_), lambda i: (0, 0)),
        ],
        out_specs=[
            pl.BlockSpec((BLK, HC), lambda i: (i, 0)),
            pl.BlockSpec((BLK, 2 * H_), lambda i: (i, 0)),
        ],
        out_shape=[
            jax.ShapeDtypeStruct((n, HC), jnp.float32),
            jax.ShapeDtypeStruct((n, 2 * H_), jnp.float32),
        ],
    )(x, W1, AW)
    al_s = al[:, :H_]
    al_d = al[:, H_:]

    loops = jnp.arange(n, dtype=edge_index.dtype)
    src = jnp.concatenate([edge_index[0], loops])
    dst = jnp.concatenate([edge_index[1], loops])

    v = al_s[src] + al_d[dst]
    alpha = jnp.where(v >= 0, v, 0.2 * v)
    amax = jax.ops.segment_max(alpha, dst, num_segments=n)
    amax = jnp.where(jnp.isfinite(amax), amax, 0.0)
    ex = jnp.exp(alpha - amax[dst])
    denom = jax.ops.segment_sum(ex, dst, num_segments=n)
    coef = ex / (denom[dst] + 1e-16)
    msg = coef[:, :, None] * h.reshape(n, H_, C)[src]
    gat = jax.ops.segment_sum(msg, dst, num_segments=n).reshape(n, HC)

    ones = jnp.ones(src.shape, jnp.float32)
    deg = jax.ops.segment_sum(ones, dst, num_segments=n)
    dinv = jnp.where(deg > 0, jax.lax.rsqrt(deg), 0.0)
    norm = dinv[src] * dinv[dst]

    h2 = pl.pallas_call(
        _gcn_mm_kernel,
        grid=grid,
        in_specs=[
            pl.BlockSpec((BLK, HC), lambda i: (i, 0)),
            pl.BlockSpec((1, HC), lambda i: (0, 0)),
            pl.BlockSpec((HC, HC), lambda i: (0, 0)),
        ],
        out_specs=pl.BlockSpec((BLK, HC), lambda i: (i, 0)),
        out_shape=jax.ShapeDtypeStruct((n, HC), jnp.float32),
    )(gat, b1.reshape(1, HC), W2)

    m2 = jax.ops.segment_sum(norm[:, None] * h2[src], dst, num_segments=n)
    hh = jnp.maximum(m2 + b2[None, :], 0.0)

    counts = jax.ops.segment_sum(jnp.ones((n,), jnp.float32), batch, num_segments=nG)
    gmean = jax.ops.segment_sum(hh, batch, num_segments=nG)
    gmean = gmean / jnp.maximum(counts, 1.0)[:, None]
    gmax = jax.ops.segment_max(hh, batch, num_segments=nG)
    gmax = jnp.where(jnp.isfinite(gmax), gmax, 0.0)
    g = jnp.concatenate([gmax, gmean], axis=1)

    out = pl.pallas_call(
        _mlp_kernel,
        in_specs=[
            pl.BlockSpec(g.shape, lambda: (0, 0)),
            pl.BlockSpec(Wf1.shape, lambda: (0, 0)),
            pl.BlockSpec((1, Wf1.shape[1]), lambda: (0, 0)),
            pl.BlockSpec(Wf2.shape, lambda: (0, 0)),
            pl.BlockSpec((1, Wf2.shape[1]), lambda: (0, 0)),
        ],
        out_specs=pl.BlockSpec((g.shape[0], Wf2.shape[1]), lambda: (0, 0)),
        out_shape=jax.ShapeDtypeStruct((g.shape[0], Wf2.shape[1]), jnp.float32),
    )(g, Wf1, bf1.reshape(1, -1), Wf2, bf2.reshape(1, -1))
    return out


# speedup vs baseline: 1.0776x; 1.0776x over previous
import jax
import jax.numpy as jnp
from jax.experimental import pallas as pl

H_ = 10


def _proj_attn_kernel(x_ref, w1_ref, aw_ref, h_ref, al_ref):
    h = jnp.dot(x_ref[...], w1_ref[...], preferred_element_type=jnp.float32)
    h_ref[...] = h
    al_ref[...] = jnp.dot(h, aw_ref[...], preferred_element_type=jnp.float32)


def _gcn_mm_kernel(g_ref, b1_ref, w2_ref, dinv_ref, out_ref):
    hin = jnp.maximum(g_ref[...] + b1_ref[...], 0.0)
    h2 = jnp.dot(hin, w2_ref[...], preferred_element_type=jnp.float32)
    out_ref[...] = h2 * dinv_ref[...]


def _mlp_kernel(g_ref, wf1_ref, bf1_ref, wf2_ref, bf2_ref, out_ref):
    t = jnp.dot(g_ref[...], wf1_ref[...], preferred_element_type=jnp.float32)
    t = jnp.maximum(t + bf1_ref[...], 0.0)
    out_ref[...] = (
        jnp.dot(t, wf2_ref[...], preferred_element_type=jnp.float32) + bf2_ref[...]
    )


def kernel(x, edge_index, batch, W1, a_src, a_dst, b1, W2, b2, Wf1, bf1, Wf2, bf2):
    n, f = x.shape
    HC = W1.shape[1]
    C = HC // H_
    nG = 256

    eye = jnp.eye(H_, dtype=x.dtype)
    As = (a_src[:, :, None] * eye[:, None, :]).reshape(HC, H_)
    Ad = (a_dst[:, :, None] * eye[:, None, :]).reshape(HC, H_)
    AW = jnp.concatenate([As, Ad], axis=1)

    BLK = 1000
    grid = (n // BLK,)
    h, al = pl.pallas_call(
        _proj_attn_kernel,
        grid=grid,
        in_specs=[
            pl.BlockSpec((BLK, f), lambda i: (i, 0)),
            pl.BlockSpec((f, HC), lambda i: (0, 0)),
            pl.BlockSpec((HC, 2 * H_), lambda i: (0, 0)),
        ],
        out_specs=[
            pl.BlockSpec((BLK, HC), lambda i: (i, 0)),
            pl.BlockSpec((BLK, 2 * H_), lambda i: (i, 0)),
        ],
        out_shape=[
            jax.ShapeDtypeStruct((n, HC), jnp.float32),
            jax.ShapeDtypeStruct((n, 2 * H_), jnp.float32),
        ],
    )(x, W1, AW)
    al_s = al[:, :H_]
    al_d = al[:, H_:]

    loops = jnp.arange(n, dtype=edge_index.dtype)
    src = jnp.concatenate([edge_index[0], loops])
    dst = jnp.concatenate([edge_index[1], loops])

    v = al_s[src] + al_d[dst]
    alpha = jnp.where(v >= 0, v, 0.2 * v)
    amax = jax.ops.segment_max(alpha, dst, num_segments=n)
    amax = jnp.where(jnp.isfinite(amax), amax, 0.0)
    ex = jnp.exp(alpha - amax[dst])
    denom = jax.ops.segment_sum(ex, dst, num_segments=n)
    coef = ex / (denom[dst] + 1e-16)
    msg = coef[:, :, None] * h.reshape(n, H_, C)[src]
    gat = jax.ops.segment_sum(msg, dst, num_segments=n).reshape(n, HC)

    ones = jnp.ones(src.shape, jnp.float32)
    deg = jax.ops.segment_sum(ones, dst, num_segments=n)
    dinv = jnp.where(deg > 0, jax.lax.rsqrt(deg), 0.0)

    h2s = pl.pallas_call(
        _gcn_mm_kernel,
        grid=grid,
        in_specs=[
            pl.BlockSpec((BLK, HC), lambda i: (i, 0)),
            pl.BlockSpec((1, HC), lambda i: (0, 0)),
            pl.BlockSpec((HC, HC), lambda i: (0, 0)),
            pl.BlockSpec((BLK, 1), lambda i: (i, 0)),
        ],
        out_specs=pl.BlockSpec((BLK, HC), lambda i: (i, 0)),
        out_shape=jax.ShapeDtypeStruct((n, HC), jnp.float32),
    )(gat, b1.reshape(1, HC), W2, dinv.reshape(n, 1))

    m2 = jax.ops.segment_sum(h2s[src], dst, num_segments=n)
    hh = jnp.maximum(dinv[:, None] * m2 + b2[None, :], 0.0)

    counts = jax.ops.segment_sum(jnp.ones((n,), jnp.float32), batch, num_segments=nG)
    gmean = jax.ops.segment_sum(hh, batch, num_segments=nG)
    gmean = gmean / jnp.maximum(counts, 1.0)[:, None]
    gmax = jax.ops.segment_max(hh, batch, num_segments=nG)
    gmax = jnp.where(jnp.isfinite(gmax), gmax, 0.0)
    g = jnp.concatenate([gmax, gmean], axis=1)

    out = pl.pallas_call(
        _mlp_kernel,
        in_specs=[
            pl.BlockSpec(g.shape, lambda: (0, 0)),
            pl.BlockSpec(Wf1.shape, lambda: (0, 0)),
            pl.BlockSpec((1, Wf1.shape[1]), lambda: (0, 0)),
            pl.BlockSpec(Wf2.shape, lambda: (0, 0)),
            pl.BlockSpec((1, Wf2.shape[1]), lambda: (0, 0)),
        ],
        out_specs=pl.BlockSpec((g.shape[0], Wf2.shape[1]), lambda: (0, 0)),
        out_shape=jax.ShapeDtypeStruct((g.shape[0], Wf2.shape[1]), jnp.float32),
    )(g, Wf1, bf1.reshape(1, -1), Wf2, bf2.reshape(1, -1))
    return out


# speedup vs baseline: 1.1126x; 1.0325x over previous
import jax
import jax.numpy as jnp
from jax.experimental import pallas as pl

H_ = 10


def _proj_attn_kernel(x_ref, w1_ref, aw_ref, h_ref, al_ref):
    h = jnp.dot(x_ref[...], w1_ref[...], preferred_element_type=jnp.float32)
    h_ref[...] = h
    al_ref[...] = jnp.dot(h, aw_ref[...], preferred_element_type=jnp.float32)


def _gcn_mm_kernel(g_ref, b1_ref, w2_ref, dinv_ref, out_ref):
    hin = jnp.maximum(g_ref[...] + b1_ref[...], 0.0)
    h2 = jnp.dot(hin, w2_ref[...], preferred_element_type=jnp.float32)
    out_ref[...] = h2 * dinv_ref[...]


def _mlp_kernel(g_ref, wf1_ref, bf1_ref, wf2_ref, bf2_ref, out_ref):
    t = jnp.dot(g_ref[...], wf1_ref[...], preferred_element_type=jnp.float32)
    t = jnp.maximum(t + bf1_ref[...], 0.0)
    out_ref[...] = (
        jnp.dot(t, wf2_ref[...], preferred_element_type=jnp.float32) + bf2_ref[...]
    )


def kernel(x, edge_index, batch, W1, a_src, a_dst, b1, W2, b2, Wf1, bf1, Wf2, bf2):
    n, f = x.shape
    HC = W1.shape[1]
    C = HC // H_
    nG = 256

    eye = jnp.eye(H_, dtype=x.dtype)
    As = (a_src[:, :, None] * eye[:, None, :]).reshape(HC, H_)
    Ad = (a_dst[:, :, None] * eye[:, None, :]).reshape(HC, H_)
    AW = jnp.concatenate([As, Ad], axis=1)

    BLK = 1000
    grid = (n // BLK,)
    h, al = pl.pallas_call(
        _proj_attn_kernel,
        grid=grid,
        in_specs=[
            pl.BlockSpec((BLK, f), lambda i: (i, 0)),
            pl.BlockSpec((f, HC), lambda i: (0, 0)),
            pl.BlockSpec((HC, 2 * H_), lambda i: (0, 0)),
        ],
        out_specs=[
            pl.BlockSpec((BLK, HC), lambda i: (i, 0)),
            pl.BlockSpec((BLK, 2 * H_), lambda i: (i, 0)),
        ],
        out_shape=[
            jax.ShapeDtypeStruct((n, HC), jnp.float32),
            jax.ShapeDtypeStruct((n, 2 * H_), jnp.float32),
        ],
    )(x, W1, AW)
    al_s = al[:, :H_]
    al_d = al[:, H_:]

    loops = jnp.arange(n, dtype=edge_index.dtype)
    src = jnp.concatenate([edge_index[0], loops])
    dst = jnp.concatenate([edge_index[1], loops])

    v = al_s[src] + al_d[dst]
    alpha = jnp.where(v >= 0, v, 0.2 * v)
    amax = jax.ops.segment_max(alpha, dst, num_segments=n)
    amax = jnp.where(jnp.isfinite(amax), amax, 0.0)
    ex = jnp.exp(alpha - amax[dst])
    denom = jax.ops.segment_sum(ex, dst, num_segments=n)
    msg = ex[:, :, None] * h.reshape(n, H_, C)[src]
    gat = jax.ops.segment_sum(msg, dst, num_segments=n)
    gat = (gat / (denom[:, :, None] + 1e-16)).reshape(n, HC)

    ones = jnp.ones(src.shape, jnp.float32)
    deg = jax.ops.segment_sum(ones, dst, num_segments=n)
    dinv = jnp.where(deg > 0, jax.lax.rsqrt(deg), 0.0)

    h2s = pl.pallas_call(
        _gcn_mm_kernel,
        grid=grid,
        in_specs=[
            pl.BlockSpec((BLK, HC), lambda i: (i, 0)),
            pl.BlockSpec((1, HC), lambda i: (0, 0)),
            pl.BlockSpec((HC, HC), lambda i: (0, 0)),
            pl.BlockSpec((BLK, 1), lambda i: (i, 0)),
        ],
        out_specs=pl.BlockSpec((BLK, HC), lambda i: (i, 0)),
        out_shape=jax.ShapeDtypeStruct((n, HC), jnp.float32),
    )(gat, b1.reshape(1, HC), W2, dinv.reshape(n, 1))

    m2 = jax.ops.segment_sum(h2s[src], dst, num_segments=n)
    hh = jnp.maximum(dinv[:, None] * m2 + b2[None, :], 0.0)

    counts = jax.ops.segment_sum(jnp.ones((n,), jnp.float32), batch, num_segments=nG)
    gmean = jax.ops.segment_sum(hh, batch, num_segments=nG)
    gmean = gmean / jnp.maximum(counts, 1.0)[:, None]
    gmax = jax.ops.segment_max(hh, batch, num_segments=nG)
    gmax = jnp.where(jnp.isfinite(gmax), gmax, 0.0)
    g = jnp.concatenate([gmax, gmean], axis=1)

    out = pl.pallas_call(
        _mlp_kernel,
        in_specs=[
            pl.BlockSpec(g.shape, lambda: (0, 0)),
            pl.BlockSpec(Wf1.shape, lambda: (0, 0)),
            pl.BlockSpec((1, Wf1.shape[1]), lambda: (0, 0)),
            pl.BlockSpec(Wf2.shape, lambda: (0, 0)),
            pl.BlockSpec((1, Wf2.shape[1]), lambda: (0, 0)),
        ],
        out_specs=pl.BlockSpec((g.shape[0], Wf2.shape[1]), lambda: (0, 0)),
        out_shape=jax.ShapeDtypeStruct((g.shape[0], Wf2.shape[1]), jnp.float32),
    )(g, Wf1, bf1.reshape(1, -1), Wf2, bf2.reshape(1, -1))
    return out
